# baseline (speedup 1.0000x reference)
import jax
import jax.numpy as jnp
from jax.experimental import pallas as pl
from jax.experimental.pallas import tpu as pltpu

_B, _N = 256, 512
_EMB, _VIS, _KPT = 128, 1, 17
_TOKEN_DIM = 128
_ROWS = 1024


def _body(mask_ref, emb_ref, vis_ref, bbox_ref, kpt_ref,
          wemb_ref, wvis_ref, wbbox_ref, wkpt_ref, ab_ref, sb_ref, out_ref):
    dn = (((1,), (1,)), ((), ()))
    acc = jax.lax.dot_general(emb_ref[...], wemb_ref[...], dn,
                              preferred_element_type=jnp.float32)
    acc += jax.lax.dot_general(kpt_ref[...], wkpt_ref[...], dn,
                               preferred_element_type=jnp.float32)
    acc += jax.lax.dot_general(bbox_ref[...], wbbox_ref[...], dn,
                               preferred_element_type=jnp.float32)
    acc += vis_ref[...] * wvis_ref[...]
    acc += ab_ref[...] + sb_ref[...]
    out_ref[...] = jnp.where(mask_ref[...], acc, 0.0)


def kernel(feats_masks, embeddings, visibility_scores, bbox_ltwh,
           keypoints_xyc, app_W, app_b, st_W, st_b):
    m = _B * _N
    mask = feats_masks.reshape(m, 1)
    emb = embeddings.reshape(m, _EMB)
    vis = visibility_scores.reshape(m, _VIS)
    bbox = bbox_ltwh.reshape(m, 4)
    kpt = keypoints_xyc.reshape(m, _KPT * 3)
    wemb = app_W[:, :_EMB]
    wvis = app_W[:, _EMB].reshape(1, _TOKEN_DIM)
    wbbox = st_W[:, :4]
    wkpt = st_W[:, 4:]
    ab = app_b.reshape(1, _TOKEN_DIM)
    sb = st_b.reshape(1, _TOKEN_DIM)

    grid = (m // _ROWS,)
    row = lambda i: (i, 0)
    rep = lambda i: (0, 0)
    out = pl.pallas_call(
        _body,
        grid=grid,
        in_specs=[
            pl.BlockSpec((_ROWS, 1), row),
            pl.BlockSpec((_ROWS, _EMB), row),
            pl.BlockSpec((_ROWS, _VIS), row),
            pl.BlockSpec((_ROWS, 4), row),
            pl.BlockSpec((_ROWS, _KPT * 3), row),
            pl.BlockSpec((_TOKEN_DIM, _EMB), rep),
            pl.BlockSpec((1, _TOKEN_DIM), rep),
            pl.BlockSpec((_TOKEN_DIM, 4), rep),
            pl.BlockSpec((_TOKEN_DIM, _KPT * 3), rep),
            pl.BlockSpec((1, _TOKEN_DIM), rep),
            pl.BlockSpec((1, _TOKEN_DIM), rep),
        ],
        out_specs=pl.BlockSpec((_ROWS, _TOKEN_DIM), row),
        out_shape=jax.ShapeDtypeStruct((m, _TOKEN_DIM), jnp.float32),
        compiler_params=pltpu.CompilerParams(
            dimension_semantics=("arbitrary",),
        ),
    )(mask, emb, vis, bbox, kpt, wemb, wvis, wbbox, wkpt, ab, sb)
    return out.reshape(_B, _N, _TOKEN_DIM)


# speedup vs baseline: 1.1492x; 1.1492x over previous
import jax
import jax.numpy as jnp
from jax.experimental import pallas as pl
from jax.experimental.pallas import tpu as pltpu

_B, _N = 256, 512
_EMB, _KPT = 128, 17
_TOKEN_DIM = 128


def _body(mask_ref, emb_ref, vis_ref, bbox_ref, kpt_ref,
          wembT_ref, wvis_ref, wbboxT_ref, wkptT_ref, ab_ref, sb_ref,
          out_ref):
    f32 = jnp.float32
    dn_t = (((0,), (0,)), ((), ()))
    acc = jax.lax.dot_general(emb_ref[...], wembT_ref[...],
                              (((1,), (0,)), ((), ())),
                              preferred_element_type=f32)
    acc += jax.lax.dot_general(kpt_ref[:, 0, 0, :], wkptT_ref[...], dn_t,
                               preferred_element_type=f32)
    ones = jnp.ones((1, _N), f32)
    xs = jnp.concatenate([bbox_ref[0][...], vis_ref[0][...], ones, ones], axis=0)
    ws = jnp.concatenate([wbboxT_ref[...], wvis_ref[...],
                          ab_ref[...], sb_ref[...]], axis=0)
    acc += jax.lax.dot_general(xs, ws, dn_t, preferred_element_type=f32)
    mask_col = jnp.transpose(mask_ref[0][...], (1, 0))
    out_ref[...] = acc * mask_col


def kernel(feats_masks, embeddings, visibility_scores, bbox_ltwh,
           keypoints_xyc, app_W, app_b, st_W, st_b):
    m = _B * _N
    maskf = feats_masks.astype(jnp.float32).reshape(_B, 1, _N)
    emb = embeddings.reshape(m, _EMB)
    vis = visibility_scores.reshape(_B, 1, _N)
    bboxT = bbox_ltwh.transpose(0, 2, 1)
    kptT = keypoints_xyc.transpose(2, 3, 0, 1).reshape(_KPT * 3, _B, 1, _N)
    app_WT = app_W.T
    wembT = app_WT[:_EMB]
    wvis = app_WT[_EMB:]
    st_WT = st_W.T
    wbboxT = st_WT[:4]
    wkptT = st_WT[4:]
    ab = app_b.reshape(1, _TOKEN_DIM)
    sb = st_b.reshape(1, _TOKEN_DIM)

    grid = (_B,)
    rep = lambda i: (0, 0)
    out = pl.pallas_call(
        _body,
        grid=grid,
        in_specs=[
            pl.BlockSpec((1, 1, _N), lambda i: (i, 0, 0)),
            pl.BlockSpec((_N, _EMB), lambda i: (i, 0)),
            pl.BlockSpec((1, 1, _N), lambda i: (i, 0, 0)),
            pl.BlockSpec((1, 4, _N), lambda i: (i, 0, 0)),
            pl.BlockSpec((_KPT * 3, 1, 1, _N), lambda i: (0, i, 0, 0)),
            pl.BlockSpec((_EMB, _TOKEN_DIM), rep),
            pl.BlockSpec((1, _TOKEN_DIM), rep),
            pl.BlockSpec((4, _TOKEN_DIM), rep),
            pl.BlockSpec((_KPT * 3, _TOKEN_DIM), rep),
            pl.BlockSpec((1, _TOKEN_DIM), rep),
            pl.BlockSpec((1, _TOKEN_DIM), rep),
        ],
        out_specs=pl.BlockSpec((_N, _TOKEN_DIM), lambda i: (i, 0)),
        out_shape=jax.ShapeDtypeStruct((m, _TOKEN_DIM), jnp.float32),
        compiler_params=pltpu.CompilerParams(
            dimension_semantics=("arbitrary",),
        ),
    )(maskf, emb, vis, bboxT, kptT, wembT, wvis, wbboxT, wkptT, ab, sb)
    return out.reshape(_B, _N, _TOKEN_DIM)


# speedup vs baseline: 1.9932x; 1.7344x over previous
import jax
import jax.numpy as jnp
from jax.experimental import pallas as pl
from jax.experimental.pallas import tpu as pltpu

_B, _N = 256, 512
_EMB, _KPT = 128, 17
_TOKEN_DIM = 128
_G = 8


def _body(mask_ref, emb_ref, vis_ref, bbox_ref, kpt_ref,
          wembT_ref, wvis_ref, wbboxT_ref, wkptT_ref, ab_ref, sb_ref,
          out_ref):
    f32 = jnp.float32
    dn_t = (((0,), (0,)), ((), ()))
    ws = jnp.concatenate([wbboxT_ref[...], wvis_ref[...],
                          ab_ref[...], sb_ref[...]], axis=0)
    ones = jnp.ones((1, _N), f32)
    mask_cols = jnp.transpose(mask_ref[:, 0, :], (1, 0))
    for j in range(_G):
        acc = jax.lax.dot_general(emb_ref[pl.ds(j * _N, _N), :],
                                  wembT_ref[...],
                                  (((1,), (0,)), ((), ())),
                                  preferred_element_type=f32)
        acc += jax.lax.dot_general(kpt_ref[:, j, 0, :], wkptT_ref[...], dn_t,
                                   preferred_element_type=f32)
        xs = jnp.concatenate([bbox_ref[j], vis_ref[j], ones, ones], axis=0)
        acc += jax.lax.dot_general(xs, ws, dn_t, preferred_element_type=f32)
        out_ref[pl.ds(j * _N, _N), :] = acc * mask_cols[:, j:j + 1]


def kernel(feats_masks, embeddings, visibility_scores, bbox_ltwh,
           keypoints_xyc, app_W, app_b, st_W, st_b):
    m = _B * _N
    maskf = feats_masks.astype(jnp.float32).reshape(_B, 1, _N)
    emb = embeddings.reshape(m, _EMB)
    vis = visibility_scores.reshape(_B, 1, _N)
    bboxT = bbox_ltwh.transpose(0, 2, 1)
    kptT = keypoints_xyc.transpose(2, 3, 0, 1).reshape(_KPT * 3, _B, 1, _N)
    app_WT = app_W.T
    wembT = app_WT[:_EMB]
    wvis = app_WT[_EMB:]
    st_WT = st_W.T
    wbboxT = st_WT[:4]
    wkptT = st_WT[4:]
    ab = app_b.reshape(1, _TOKEN_DIM)
    sb = st_b.reshape(1, _TOKEN_DIM)

    grid = (_B // _G,)
    rep = lambda i: (0, 0)
    out = pl.pallas_call(
        _body,
        grid=grid,
        in_specs=[
            pl.BlockSpec((_G, 1, _N), lambda i: (i, 0, 0)),
            pl.BlockSpec((_G * _N, _EMB), lambda i: (i, 0)),
            pl.BlockSpec((_G, 1, _N), lambda i: (i, 0, 0)),
            pl.BlockSpec((_G, 4, _N), lambda i: (i, 0, 0)),
            pl.BlockSpec((_KPT * 3, _G, 1, _N), lambda i: (0, i, 0, 0)),
            pl.BlockSpec((_EMB, _TOKEN_DIM), rep),
            pl.BlockSpec((1, _TOKEN_DIM), rep),
            pl.BlockSpec((4, _TOKEN_DIM), rep),
            pl.BlockSpec((_KPT * 3, _TOKEN_DIM), rep),
            pl.BlockSpec((1, _TOKEN_DIM), rep),
            pl.BlockSpec((1, _TOKEN_DIM), rep),
        ],
        out_specs=pl.BlockSpec((_G * _N, _TOKEN_DIM), lambda i: (i, 0)),
        out_shape=jax.ShapeDtypeStruct((m, _TOKEN_DIM), jnp.float32),
        compiler_params=pltpu.CompilerParams(
            dimension_semantics=("arbitrary",),
        ),
    )(maskf, emb, vis, bboxT, kptT, wembT, wvis, wbboxT, wkptT, ab, sb)
    return out.reshape(_B, _N, _TOKEN_DIM)


# speedup vs baseline: 2.1095x; 1.0583x over previous
import jax
import jax.numpy as jnp
from jax.experimental import pallas as pl
from jax.experimental.pallas import tpu as pltpu

_B, _N = 256, 512
_EMB, _KPT = 128, 17
_TOKEN_DIM = 128
_G = 16


def _body(mask_ref, emb_ref, vis_ref, bbox_ref, kpt_ref,
          wembT_ref, wvis_ref, wbboxT_ref, wkptT_ref, ab_ref, sb_ref,
          out_ref):
    f32 = jnp.float32
    dn_t = (((0,), (0,)), ((), ()))
    ws = jnp.concatenate([wbboxT_ref[...], wvis_ref[...],
                          ab_ref[...], sb_ref[...]], axis=0)
    ones = jnp.ones((1, _N), f32)
    mask_cols = jnp.transpose(mask_ref[:, 0, :], (1, 0))
    for j in range(_G):
        acc = jax.lax.dot_general(emb_ref[pl.ds(j * _N, _N), :],
                                  wembT_ref[...],
                                  (((1,), (0,)), ((), ())),
                                  preferred_element_type=f32)
        acc += jax.lax.dot_general(kpt_ref[:, j, 0, :], wkptT_ref[...], dn_t,
                                   preferred_element_type=f32)
        xs = jnp.concatenate([bbox_ref[j], vis_ref[j], ones, ones], axis=0)
        acc += jax.lax.dot_general(xs, ws, dn_t, preferred_element_type=f32)
        out_ref[pl.ds(j * _N, _N), :] = acc * mask_cols[:, j:j + 1]


def kernel(feats_masks, embeddings, visibility_scores, bbox_ltwh,
           keypoints_xyc, app_W, app_b, st_W, st_b):
    m = _B * _N
    maskf = feats_masks.astype(jnp.float32).reshape(_B, 1, _N)
    emb = embeddings.reshape(m, _EMB)
    vis = visibility_scores.reshape(_B, 1, _N)
    bboxT = bbox_ltwh.transpose(0, 2, 1)
    kptT = keypoints_xyc.transpose(2, 3, 0, 1).reshape(_KPT * 3, _B, 1, _N)
    app_WT = app_W.T
    wembT = app_WT[:_EMB]
    wvis = app_WT[_EMB:]
    st_WT = st_W.T
    wbboxT = st_WT[:4]
    wkptT = st_WT[4:]
    ab = app_b.reshape(1, _TOKEN_DIM)
    sb = st_b.reshape(1, _TOKEN_DIM)

    grid = (_B // _G,)
    rep = lambda i: (0, 0)
    out = pl.pallas_call(
        _body,
        grid=grid,
        in_specs=[
            pl.BlockSpec((_G, 1, _N), lambda i: (i, 0, 0)),
            pl.BlockSpec((_G * _N, _EMB), lambda i: (i, 0)),
            pl.BlockSpec((_G, 1, _N), lambda i: (i, 0, 0)),
            pl.BlockSpec((_G, 4, _N), lambda i: (i, 0, 0)),
            pl.BlockSpec((_KPT * 3, _G, 1, _N), lambda i: (0, i, 0, 0)),
            pl.BlockSpec((_EMB, _TOKEN_DIM), rep),
            pl.BlockSpec((1, _TOKEN_DIM), rep),
            pl.BlockSpec((4, _TOKEN_DIM), rep),
            pl.BlockSpec((_KPT * 3, _TOKEN_DIM), rep),
            pl.BlockSpec((1, _TOKEN_DIM), rep),
            pl.BlockSpec((1, _TOKEN_DIM), rep),
        ],
        out_specs=pl.BlockSpec((_G * _N, _TOKEN_DIM), lambda i: (i, 0)),
        out_shape=jax.ShapeDtypeStruct((m, _TOKEN_DIM), jnp.float32),
        compiler_params=pltpu.CompilerParams(
            dimension_semantics=("arbitrary",),
        ),
    )(maskf, emb, vis, bboxT, kptT, wembT, wvis, wbboxT, wkptT, ab, sb)
    return out.reshape(_B, _N, _TOKEN_DIM)


# speedup vs baseline: 2.1421x; 1.0155x over previous
import jax
import jax.numpy as jnp
from jax.experimental import pallas as pl
from jax.experimental.pallas import tpu as pltpu

_B, _N = 256, 512
_EMB, _KPT = 128, 17
_TOKEN_DIM = 128
_G = 32


def _body(mask_ref, emb_ref, vis_ref, bbox_ref, kpt_ref,
          wembT_ref, wvis_ref, wbboxT_ref, wkptT_ref, ab_ref, sb_ref,
          out_ref):
    f32 = jnp.float32
    dn_t = (((0,), (0,)), ((), ()))
    ws = jnp.concatenate([wbboxT_ref[...], wvis_ref[...],
                          ab_ref[...], sb_ref[...]], axis=0)
    ones = jnp.ones((1, _N), f32)
    mask_cols = jnp.transpose(mask_ref[:, 0, :], (1, 0))
    for j in range(_G):
        acc = jax.lax.dot_general(emb_ref[pl.ds(j * _N, _N), :],
                                  wembT_ref[...],
                                  (((1,), (0,)), ((), ())),
                                  preferred_element_type=f32)
        acc += jax.lax.dot_general(kpt_ref[:, j, 0, :], wkptT_ref[...], dn_t,
                                   preferred_element_type=f32)
        xs = jnp.concatenate([bbox_ref[j], vis_ref[j], ones, ones], axis=0)
        acc += jax.lax.dot_general(xs, ws, dn_t, preferred_element_type=f32)
        out_ref[pl.ds(j * _N, _N), :] = acc * mask_cols[:, j:j + 1]


def kernel(feats_masks, embeddings, visibility_scores, bbox_ltwh,
           keypoints_xyc, app_W, app_b, st_W, st_b):
    m = _B * _N
    maskf = feats_masks.astype(jnp.float32).reshape(_B, 1, _N)
    emb = embeddings.reshape(m, _EMB)
    vis = visibility_scores.reshape(_B, 1, _N)
    bboxT = bbox_ltwh.transpose(0, 2, 1)
    kptT = keypoints_xyc.transpose(2, 3, 0, 1).reshape(_KPT * 3, _B, 1, _N)
    app_WT = app_W.T
    wembT = app_WT[:_EMB]
    wvis = app_WT[_EMB:]
    st_WT = st_W.T
    wbboxT = st_WT[:4]
    wkptT = st_WT[4:]
    ab = app_b.reshape(1, _TOKEN_DIM)
    sb = st_b.reshape(1, _TOKEN_DIM)

    grid = (_B // _G,)
    rep = lambda i: (0, 0)
    out = pl.pallas_call(
        _body,
        grid=grid,
        in_specs=[
            pl.BlockSpec((_G, 1, _N), lambda i: (i, 0, 0)),
            pl.BlockSpec((_G * _N, _EMB), lambda i: (i, 0)),
            pl.BlockSpec((_G, 1, _N), lambda i: (i, 0, 0)),
            pl.BlockSpec((_G, 4, _N), lambda i: (i, 0, 0)),
            pl.BlockSpec((_KPT * 3, _G, 1, _N), lambda i: (0, i, 0, 0)),
            pl.BlockSpec((_EMB, _TOKEN_DIM), rep),
            pl.BlockSpec((1, _TOKEN_DIM), rep),
            pl.BlockSpec((4, _TOKEN_DIM), rep),
            pl.BlockSpec((_KPT * 3, _TOKEN_DIM), rep),
            pl.BlockSpec((1, _TOKEN_DIM), rep),
            pl.BlockSpec((1, _TOKEN_DIM), rep),
        ],
        out_specs=pl.BlockSpec((_G * _N, _TOKEN_DIM), lambda i: (i, 0)),
        out_shape=jax.ShapeDtypeStruct((m, _TOKEN_DIM), jnp.float32),
        compiler_params=pltpu.CompilerParams(
            dimension_semantics=("arbitrary",),
        ),
    )(maskf, emb, vis, bboxT, kptT, wembT, wvis, wbboxT, wkptT, ab, sb)
    return out.reshape(_B, _N, _TOKEN_DIM)


# speedup vs baseline: 2.1460x; 1.0018x over previous
import jax
import jax.numpy as jnp
from jax.experimental import pallas as pl
from jax.experimental.pallas import tpu as pltpu

_B, _N = 256, 512
_EMB, _KPT = 128, 17
_TOKEN_DIM = 128
_G = 32


def _body(mask_ref, emb_ref, vis_ref, bbox_ref, kpt_ref,
          wembT_ref, wvis_ref, wbboxT_ref, wkptT_ref, ab_ref, sb_ref,
          out_ref):
    f32 = jnp.float32
    dn_t = (((0,), (0,)), ((), ()))
    ws = jnp.concatenate([wbboxT_ref[...], wvis_ref[...],
                          ab_ref[...], sb_ref[...]], axis=0)
    ones = jnp.ones((1, _N), f32)
    mask_cols = jnp.transpose(mask_ref[:, 0, :], (1, 0))
    for j in range(_G):
        acc = jax.lax.dot_general(emb_ref[pl.ds(j * _N, _N), :],
                                  wembT_ref[...],
                                  (((1,), (0,)), ((), ())),
                                  preferred_element_type=f32)
        acc += jax.lax.dot_general(kpt_ref[:, j, 0, :], wkptT_ref[...], dn_t,
                                   preferred_element_type=f32)
        xs = jnp.concatenate([bbox_ref[j], vis_ref[j], ones, ones], axis=0)
        acc += jax.lax.dot_general(xs, ws, dn_t, preferred_element_type=f32)
        out_ref[pl.ds(j * _N, _N), :] = acc * mask_cols[:, j:j + 1]


def kernel(feats_masks, embeddings, visibility_scores, bbox_ltwh,
           keypoints_xyc, app_W, app_b, st_W, st_b):
    m = _B * _N
    maskf = feats_masks.astype(jnp.float32).reshape(_B, 1, _N)
    emb = embeddings.reshape(m, _EMB)
    vis = visibility_scores.reshape(_B, 1, _N)
    bboxT = bbox_ltwh.transpose(0, 2, 1)
    kptT = keypoints_xyc.transpose(2, 3, 0, 1).reshape(_KPT * 3, _B, 1, _N)
    app_WT = app_W.T
    wembT = app_WT[:_EMB]
    wvis = app_WT[_EMB:]
    st_WT = st_W.T
    wbboxT = st_WT[:4]
    wkptT = st_WT[4:]
    ab = app_b.reshape(1, _TOKEN_DIM)
    sb = st_b.reshape(1, _TOKEN_DIM)

    grid = (_B // _G,)
    rep = lambda i: (0, 0)
    out = pl.pallas_call(
        _body,
        grid=grid,
        in_specs=[
            pl.BlockSpec((_G, 1, _N), lambda i: (i, 0, 0)),
            pl.BlockSpec((_G * _N, _EMB), lambda i: (i, 0)),
            pl.BlockSpec((_G, 1, _N), lambda i: (i, 0, 0)),
            pl.BlockSpec((_G, 4, _N), lambda i: (i, 0, 0)),
            pl.BlockSpec((_KPT * 3, _G, 1, _N), lambda i: (0, i, 0, 0)),
            pl.BlockSpec((_EMB, _TOKEN_DIM), rep),
            pl.BlockSpec((1, _TOKEN_DIM), rep),
            pl.BlockSpec((4, _TOKEN_DIM), rep),
            pl.BlockSpec((_KPT * 3, _TOKEN_DIM), rep),
            pl.BlockSpec((1, _TOKEN_DIM), rep),
            pl.BlockSpec((1, _TOKEN_DIM), rep),
        ],
        out_specs=pl.BlockSpec((_G * _N, _TOKEN_DIM), lambda i: (i, 0)),
        out_shape=jax.ShapeDtypeStruct((m, _TOKEN_DIM), jnp.float32),
        compiler_params=pltpu.CompilerParams(
            dimension_semantics=("parallel",),
        ),
    )(maskf, emb, vis, bboxT, kptT, wembT, wvis, wbboxT, wkptT, ab, sb)
    return out.reshape(_B, _N, _TOKEN_DIM)
